# Initial kernel scaffold; baseline (speedup 1.0000x reference)
import jax
import jax.numpy as jnp
from jax.experimental import pallas as pl


def kernel(x, mask, predict, table):
    raise NotImplementedError("write your pallas kernel here")


# speedup vs baseline: 1.6996x; 1.6996x over previous
import functools

import jax
import jax.numpy as jnp
from jax import lax
from jax.experimental import pallas as pl
from jax.experimental.pallas import tpu as pltpu
from jax.experimental.pallas import tpu_sc as plsc

_LANES = 16
_NC, _NS = 2, 16
_NW = _NC * _NS
_GSZ = 128


@functools.lru_cache(maxsize=None)
def _build(n, d, chunk):
    n_per_w = n // _NW
    n_chunks = n_per_w // chunk
    n_g = chunk // _GSZ
    groups = chunk // _LANES
    mesh = plsc.VectorSubcoreMesh(core_axis_name="c", subcore_axis_name="s")

    @functools.partial(
        pl.kernel,
        out_type=jax.ShapeDtypeStruct((n, d), jnp.float32),
        mesh=mesh,
        scratch_types=[
            pltpu.VMEM((chunk,), jnp.int32),
            pltpu.VMEM((chunk,), jnp.int32),
            pltpu.VMEM((n_g, _GSZ), jnp.int32),
            pltpu.VMEM((chunk, d), jnp.float32),
            pltpu.SemaphoreType.DMA,
        ],
        compiler_params=pltpu.CompilerParams(use_tc_tiling_on_sc=False),
    )
    def emb(x_hbm, m_hbm, table_hbm, out_hbm, x_v, m_v, idx_v, rows_v, sem):
        wid = lax.axis_index("s") * _NC + lax.axis_index("c")
        base = wid * n_per_w

        @pl.loop(0, n_chunks)
        def _chunk(c):
            row0 = base + c * chunk
            pltpu.sync_copy(x_hbm.at[pl.ds(row0, chunk)], x_v)
            pltpu.sync_copy(m_hbm.at[pl.ds(row0, chunk)], m_v)
            for g in range(groups):
                xg = x_v[pl.ds(g * _LANES, _LANES)]
                mg = m_v[pl.ds(g * _LANES, _LANES)]
                j, col = divmod(g * _LANES, _GSZ)
                idx_v[j, pl.ds(col, _LANES)] = xg * mg
            copies = [
                pltpu.async_copy(
                    table_hbm.at[idx_v.at[j]],
                    rows_v.at[pl.ds(j * _GSZ, _GSZ), :],
                    sem,
                )
                for j in range(n_g)
            ]
            for cp in copies:
                cp.wait()

            @pl.loop(0, groups)
            def _grp(g):
                m16 = m_v[pl.ds(g * _LANES, _LANES)].astype(jnp.float32)
                for r in range(_LANES):
                    mvec = m16.at[jnp.full((_LANES,), r, jnp.int32)].get(
                        mode="promise_in_bounds"
                    )
                    row = g * _LANES + r
                    for col in range(0, d, _LANES):
                        rows_v[row, pl.ds(col, _LANES)] = (
                            rows_v[row, pl.ds(col, _LANES)] * mvec
                        )

            pltpu.sync_copy(rows_v, out_hbm.at[pl.ds(row0, chunk), :])

    return emb


def kernel(x, mask, predict, table):
    b, l = x.shape
    d = table.shape[1]
    n = b * l
    out = _build(n, d, 640)(x.reshape(n), mask.reshape(n), table)
    return out.reshape(b, l, d)


# speedup vs baseline: 1.7002x; 1.0003x over previous
import functools

import jax
import jax.numpy as jnp
from jax import lax
from jax.experimental import pallas as pl
from jax.experimental.pallas import tpu as pltpu
from jax.experimental.pallas import tpu_sc as plsc

_LANES = 16
_NC, _NS = 2, 16
_NW = _NC * _NS
_GSZ = 128


@functools.lru_cache(maxsize=None)
def _build(n, d, chunk):
    n_per_w = n // _NW
    n_chunks = n_per_w // chunk
    n_g = chunk // _GSZ
    groups = chunk // _LANES
    mesh = plsc.VectorSubcoreMesh(core_axis_name="c", subcore_axis_name="s")

    @functools.partial(
        pl.kernel,
        out_type=jax.ShapeDtypeStruct((n, d), jnp.float32),
        mesh=mesh,
        scratch_types=[
            pltpu.VMEM((chunk,), jnp.int32),
            pltpu.VMEM((chunk,), jnp.int32),
            pltpu.VMEM((n_g, _GSZ), jnp.int32),
            pltpu.VMEM((chunk, d), jnp.float32),
            pltpu.SemaphoreType.DMA,
        ],
        compiler_params=pltpu.CompilerParams(use_tc_tiling_on_sc=False),
    )
    def emb(x_hbm, m_hbm, table_hbm, out_hbm, x_v, m_v, idx_v, rows_v, sem):
        wid = lax.axis_index("s") * _NC + lax.axis_index("c")
        base = wid * n_per_w

        @pl.loop(0, n_chunks)
        def _chunk(c):
            row0 = base + c * chunk
            with jax.named_scope("in_dma"):
                pltpu.sync_copy(x_hbm.at[pl.ds(row0, chunk)], x_v)
                pltpu.sync_copy(m_hbm.at[pl.ds(row0, chunk)], m_v)
            with jax.named_scope("idx_compute"):
                for g in range(groups):
                    xg = x_v[pl.ds(g * _LANES, _LANES)]
                    mg = m_v[pl.ds(g * _LANES, _LANES)]
                    j, col = divmod(g * _LANES, _GSZ)
                    idx_v[j, pl.ds(col, _LANES)] = xg * mg
            with jax.named_scope("gather"):
                copies = [
                    pltpu.async_copy(
                        table_hbm.at[idx_v.at[j]],
                        rows_v.at[pl.ds(j * _GSZ, _GSZ), :],
                        sem,
                    )
                    for j in range(n_g)
                ]
                for cp in copies:
                    cp.wait()

            with jax.named_scope("mask_mul"):

                @pl.loop(0, groups)
                def _grp(g):
                    m16 = m_v[pl.ds(g * _LANES, _LANES)].astype(jnp.float32)
                    for r in range(_LANES):
                        mvec = m16.at[jnp.full((_LANES,), r, jnp.int32)].get(
                            mode="promise_in_bounds"
                        )
                        row = g * _LANES + r
                        for col in range(0, d, _LANES):
                            rows_v[row, pl.ds(col, _LANES)] = (
                                rows_v[row, pl.ds(col, _LANES)] * mvec
                            )

            with jax.named_scope("out_dma"):
                pltpu.sync_copy(rows_v, out_hbm.at[pl.ds(row0, chunk), :])

    return emb


def kernel(x, mask, predict, table):
    b, l = x.shape
    d = table.shape[1]
    n = b * l
    out = _build(n, d, 640)(x.reshape(n), mask.reshape(n), table)
    return out.reshape(b, l, d)


# speedup vs baseline: 1.7058x; 1.0033x over previous
import functools

import jax
import jax.numpy as jnp
from jax import lax
from jax.experimental import pallas as pl
from jax.experimental.pallas import tpu as pltpu
from jax.experimental.pallas import tpu_sc as plsc

_LANES = 16
_NC, _NS = 2, 16
_NW = _NC * _NS


@functools.lru_cache(maxsize=None)
def _build(n, d, chunk):
    n_per_w = n // _NW
    n_chunks = n_per_w // chunk
    groups = chunk // _LANES
    mesh = plsc.VectorSubcoreMesh(core_axis_name="c", subcore_axis_name="s")

    @functools.partial(
        pl.kernel,
        out_type=jax.ShapeDtypeStruct((n, d), jnp.float32),
        mesh=mesh,
        scratch_types=[
            pltpu.VMEM((n_per_w,), jnp.int32),
            pltpu.VMEM((n_per_w,), jnp.int32),
            pltpu.VMEM((2, chunk, d), jnp.float32),
            pltpu.SemaphoreType.DMA,
            pltpu.SemaphoreType.DMA,
            pltpu.SemaphoreType.DMA,
            pltpu.SemaphoreType.DMA,
        ],
        compiler_params=pltpu.CompilerParams(use_tc_tiling_on_sc=False),
    )
    def emb(x_hbm, m_hbm, table_hbm, out_hbm, idx_v, m_v, rows_v, sg0, sg1, so0, so1):
        sem_g = (sg0, sg1)
        sem_o = (so0, so1)
        wid = lax.axis_index("s") * _NC + lax.axis_index("c")
        base = wid * n_per_w

        pltpu.sync_copy(x_hbm.at[pl.ds(base, n_per_w)], idx_v)
        pltpu.sync_copy(m_hbm.at[pl.ds(base, n_per_w)], m_v)

        @pl.loop(0, n_per_w // _LANES)
        def _idx(g):
            sl = pl.ds(g * _LANES, _LANES)
            idx_v[sl] = idx_v[sl] * m_v[sl]

        def fire_gather(c, b):
            return pltpu.async_copy(
                table_hbm.at[idx_v.at[pl.ds(c * chunk, chunk)]],
                rows_v.at[b],
                sem_g[b],
            )

        def mask_mul(c, b):
            @pl.loop(0, groups)
            def _grp(g):
                m16 = m_v[pl.ds(c * chunk + g * _LANES, _LANES)].astype(jnp.float32)
                for r in range(_LANES):
                    mvec = m16.at[jnp.full((_LANES,), r, jnp.int32)].get(
                        mode="promise_in_bounds"
                    )
                    row = g * _LANES + r
                    for col in range(0, d, _LANES):
                        rows_v[b, row, pl.ds(col, _LANES)] = (
                            rows_v[b, row, pl.ds(col, _LANES)] * mvec
                        )

        gather_cp = [None, None]
        out_cp = [None, None]
        gather_cp[0] = fire_gather(0, 0)
        for c in range(n_chunks):
            b = c % 2
            nb = (c + 1) % 2
            if c + 1 < n_chunks:
                if out_cp[nb] is not None:
                    out_cp[nb].wait()
                    out_cp[nb] = None
                gather_cp[nb] = fire_gather(c + 1, nb)
            gather_cp[b].wait()
            mask_mul(c, b)
            out_cp[b] = pltpu.async_copy(
                rows_v.at[b],
                out_hbm.at[pl.ds(base + c * chunk, chunk), :],
                sem_o[b],
            )
        for cp in out_cp:
            if cp is not None:
                cp.wait()

    return emb


def kernel(x, mask, predict, table):
    b, l = x.shape
    d = table.shape[1]
    n = b * l
    out = _build(n, d, 800)(x.reshape(n), mask.reshape(n), table)
    return out.reshape(b, l, d)


# speedup vs baseline: 5.8104x; 3.4062x over previous
import functools

import jax
import jax.numpy as jnp
from jax import lax
from jax.experimental import pallas as pl
from jax.experimental.pallas import tpu as pltpu
from jax.experimental.pallas import tpu_sc as plsc

_LANES = 16
_NC, _NS = 2, 16
_NW = _NC * _NS


@functools.lru_cache(maxsize=None)
def _build(n, d, chunk):
    n_per_w = n // _NW
    n_chunks = n_per_w // chunk
    groups = chunk // _LANES
    mesh = plsc.VectorSubcoreMesh(core_axis_name="c", subcore_axis_name="s")

    @functools.partial(
        pl.kernel,
        out_type=jax.ShapeDtypeStruct((n, d), jnp.float32),
        mesh=mesh,
        scratch_types=[
            pltpu.VMEM((n_per_w,), jnp.int32),
            pltpu.VMEM((n_per_w,), jnp.int32),
            pltpu.VMEM((2, chunk, d), jnp.float32),
            pltpu.SemaphoreType.DMA,
            pltpu.SemaphoreType.DMA,
            pltpu.SemaphoreType.DMA,
            pltpu.SemaphoreType.DMA,
        ],
        compiler_params=pltpu.CompilerParams(use_tc_tiling_on_sc=False),
    )
    def emb(x_hbm, m_hbm, table_hbm, out_hbm, idx_v, m_v, rows_v, sg0, sg1, so0, so1):
        sem_g = (sg0, sg1)
        sem_o = (so0, so1)
        wid = lax.axis_index("s") * _NC + lax.axis_index("c")
        base = wid * n_per_w

        pltpu.sync_copy(x_hbm.at[pl.ds(base, n_per_w)], idx_v)
        pltpu.sync_copy(m_hbm.at[pl.ds(base, n_per_w)], m_v)

        def fire_gather(c, b):
            return pltpu.async_copy(
                table_hbm.at[idx_v.at[pl.ds(c * chunk, chunk)]],
                rows_v.at[b],
                sem_g[b],
            )

        def mask_mul(c, b):
            @pl.loop(0, groups)
            def _grp(g):
                m16 = m_v[pl.ds(c * chunk + g * _LANES, _LANES)].astype(jnp.float32)
                for r in range(_LANES):
                    mvec = m16.at[jnp.full((_LANES,), r, jnp.int32)].get(
                        mode="promise_in_bounds"
                    )
                    row = g * _LANES + r
                    for col in range(0, d, _LANES):
                        rows_v[b, row, pl.ds(col, _LANES)] = (
                            rows_v[b, row, pl.ds(col, _LANES)] * mvec
                        )

        gather_cp = [None, None]
        out_cp = [None, None]
        gather_cp[0] = fire_gather(0, 0)
        for c in range(n_chunks):
            b = c % 2
            nb = (c + 1) % 2
            if c + 1 < n_chunks:
                if out_cp[nb] is not None:
                    out_cp[nb].wait()
                    out_cp[nb] = None
                gather_cp[nb] = fire_gather(c + 1, nb)
            gather_cp[b].wait()
            mask_mul(c, b)
            out_cp[b] = pltpu.async_copy(
                rows_v.at[b],
                out_hbm.at[pl.ds(base + c * chunk, chunk), :],
                sem_o[b],
            )
        for cp in out_cp:
            if cp is not None:
                cp.wait()

    return emb


def kernel(x, mask, predict, table):
    b, l = x.shape
    d = table.shape[1]
    n = b * l
    out = _build(n, d, 800)(x.reshape(n), mask.reshape(n), table)
    return out.reshape(b, l, d)
